# baseline (speedup 1.0000x reference)
import dataclasses

import jax
import jax.numpy as jnp
from jax import lax
from jax.experimental import pallas as pl
from jax.experimental.pallas import tpu as pltpu
from jax.experimental.pallas import tpu_sc as plsc

NC = 2
NS = 16
NW = NC * NS
L = 16
K = 128
D = 128


def _compiler_params():
    cp = pltpu.CompilerParams()
    if "needs_layout_passes" in pltpu.CompilerParams.__dataclass_fields__:
        cp = dataclasses.replace(cp, needs_layout_passes=False)
    return cp


_MESH = plsc.VectorSubcoreMesh(core_axis_name="c", subcore_axis_name="s")


def _make_feats(vocab, nchunks):
    rows = NW * nchunks * K

    def body(emb_hbm, ids_hbm, out_hbm, ids_v, rv0, rv1, sem0, sem1):
        c = lax.axis_index("c")
        s = lax.axis_index("s")
        w = s * NC + c
        base = w * nchunks * K
        pltpu.sync_copy(ids_hbm.at[w], ids_v)

        @pl.loop(0, nchunks, step=2)
        def _(j):
            ha = pltpu.async_copy(emb_hbm.at[ids_v.at[j]], rv0, sem0)
            hb = pltpu.async_copy(emb_hbm.at[ids_v.at[j + 1]], rv1, sem1)
            ha.wait()
            pltpu.sync_copy(rv0, out_hbm.at[pl.ds(base + j * K, K)])
            hb.wait()
            pltpu.sync_copy(rv1, out_hbm.at[pl.ds(base + (j + 1) * K, K)])

    return pl.kernel(
        body,
        out_type=jax.ShapeDtypeStruct((rows, D), jnp.float32),
        mesh=_MESH,
        scratch_types=[
            pltpu.VMEM((nchunks, K), jnp.int32),
            pltpu.VMEM((K, D), jnp.float32),
            pltpu.VMEM((K, D), jnp.float32),
            pltpu.SemaphoreType.DMA,
            pltpu.SemaphoreType.DMA,
        ],
        compiler_params=_compiler_params(),
    )


def _make_agg(nchunks, table_rows, npad):
    assert nchunks % 2 == 0
    rows_per_tile = npad // NS

    def body(table_hbm, edge_hbm, out_hbm,
             edge_v, ib0, ib1, db, rv0, rv1, acc, sem0, sem1):
        c = lax.axis_index("c")
        s = lax.axis_index("s")
        w = s * NC + c

        pltpu.sync_copy(edge_hbm.at[w], edge_v)

        def unpack(j, ib, b):
            @pl.loop(0, K, step=L)
            def _(i):
                word = edge_v[j, pl.ds(i, L)]
                ib[pl.ds(i, L)] = lax.bitwise_and(word, 0xFFFF)
                db[b, pl.ds(i, L)] = lax.shift_right_logical(word, 16)

        def start_gather(j, ib, b, rv, sem):
            unpack(j, ib, b)
            return pltpu.async_copy(table_hbm.at[ib], rv, sem)

        @pl.loop(0, K)
        def _(i):
            @pl.loop(0, D, step=L)
            def _(jj):
                rv1[i, pl.ds(jj, L)] = jnp.zeros((L,), jnp.float32)

        @pl.loop(0, rows_per_tile, step=K)
        def _(r):
            pltpu.sync_copy(rv1, acc.at[pl.ds(s * rows_per_tile + r, K)])

        plsc.subcore_barrier()

        @pl.loop(0, nchunks, step=2)
        def _(j):
            ha = start_gather(j, ib0, 0, rv0, sem0)
            hb = start_gather(j + 1, ib1, 1, rv1, sem1)
            ha.wait()
            pltpu.sync_copy(rv0, acc.at[db.at[0]], add=True)
            hb.wait()
            pltpu.sync_copy(rv1, acc.at[db.at[1]], add=True)

        plsc.subcore_barrier()
        pltpu.sync_copy(
            acc.at[pl.ds(s * rows_per_tile, rows_per_tile)],
            out_hbm.at[c, pl.ds(s * rows_per_tile, rows_per_tile)],
        )

    return pl.kernel(
        body,
        out_type=jax.ShapeDtypeStruct((NC, npad, D), jnp.float32),
        mesh=_MESH,
        scratch_types=[
            pltpu.VMEM((nchunks, K), jnp.int32),
            pltpu.VMEM((K,), jnp.int32),
            pltpu.VMEM((K,), jnp.int32),
            pltpu.VMEM((2, K), jnp.int32),
            pltpu.VMEM((K, D), jnp.float32),
            pltpu.VMEM((K, D), jnp.float32),
            pltpu.VMEM_SHARED((npad, D), jnp.float32),
            pltpu.SemaphoreType.DMA,
            pltpu.SemaphoreType.DMA,
        ],
        compiler_params=_compiler_params(),
    )


def _tc_linear_relu(p, w, b, npad):
    br = 1024

    def body(p_ref, w_ref, b_ref, o_ref):
        x = p_ref[0] + p_ref[1]
        y = lax.dot_general(
            x, w_ref[...], (((1,), (1,)), ((), ())),
            preferred_element_type=jnp.float32,
        )
        o_ref[...] = jnp.maximum(y + b_ref[...], 0.0)

    return pl.pallas_call(
        body,
        grid=(npad // br,),
        in_specs=[
            pl.BlockSpec((NC, br, D), lambda i: (0, i, 0)),
            pl.BlockSpec((D, D), lambda i: (0, 0)),
            pl.BlockSpec((1, D), lambda i: (0, 0)),
        ],
        out_specs=pl.BlockSpec((br, D), lambda i: (i, 0)),
        out_shape=jax.ShapeDtypeStruct((npad, D), jnp.float32),
    )(p, w, b)


def kernel(cncpt_ids, edge_index, emb_table, W1, b1, W2, b2):
    n = cncpt_ids.shape[0]
    e = edge_index.shape[1]
    vocab = emb_table.shape[0]

    npad = -(-(n + 1) // (NS * K)) * (NS * K)
    nchunks = -(-e // (NW * K))
    nchunks += nchunks % 2
    epad = NW * nchunks * K

    src = edge_index[0].astype(jnp.int32)
    dst = edge_index[1].astype(jnp.int32)
    packed = jnp.bitwise_or(src, jnp.left_shift(dst, 16))
    packed = jnp.concatenate(
        [packed, jnp.full((epad - e,), n << 16, jnp.int32)])
    packed = packed.reshape(NW, nchunks, K)

    fchunks = -(-n // (NW * K))
    fchunks += fchunks % 2
    frows = NW * fchunks * K
    cids = jnp.concatenate(
        [cncpt_ids.astype(jnp.int32), jnp.zeros((frows - n,), jnp.int32)])
    cids = cids.reshape(NW, fchunks, K)
    feats = _make_feats(vocab, fchunks)(emb_table, cids)

    p1 = _make_agg(nchunks, frows, npad)(feats, packed)
    h1 = _tc_linear_relu(p1, W1, b1.reshape(1, D), npad)

    p2 = _make_agg(nchunks, npad, npad)(h1, packed)
    h2 = _tc_linear_relu(p2, W2, b2.reshape(1, D), npad)
    return h2[:n]


# speedup vs baseline: 1.6059x; 1.6059x over previous
import dataclasses

import jax
import jax.numpy as jnp
from jax import lax
from jax.experimental import pallas as pl
from jax.experimental.pallas import tpu as pltpu
from jax.experimental.pallas import tpu_sc as plsc

NC = 2
NS = 16
NW = NC * NS
L = 16
K = 128
D = 128


def _compiler_params():
    cp = pltpu.CompilerParams()
    if "needs_layout_passes" in pltpu.CompilerParams.__dataclass_fields__:
        cp = dataclasses.replace(cp, needs_layout_passes=False)
    return cp


_MESH = plsc.VectorSubcoreMesh(core_axis_name="c", subcore_axis_name="s")


def _make_feats(vocab, nchunks):
    rows = NW * nchunks * K

    def body(emb_hbm, ids_hbm, out_hbm, ids_v, rv, sem):
        c = lax.axis_index("c")
        s = lax.axis_index("s")
        w = s * NC + c
        base = w * nchunks * K
        pltpu.sync_copy(ids_hbm.at[w], ids_v)

        @pl.loop(0, nchunks)
        def _(j):
            pltpu.async_copy(emb_hbm.at[ids_v.at[j]], rv, sem).wait()
            pltpu.sync_copy(rv, out_hbm.at[pl.ds(base + j * K, K)])

    return pl.kernel(
        body,
        out_type=jax.ShapeDtypeStruct((rows, D), jnp.float32),
        mesh=_MESH,
        scratch_types=[
            pltpu.VMEM((nchunks, K), jnp.int32),
            pltpu.VMEM((K, D), jnp.float32),
            pltpu.SemaphoreType.DMA,
        ],
        compiler_params=_compiler_params(),
    )


def _make_agg(nchunks, npad):
    rows_per_tile = npad // NS

    def body(table_hbm, src_hbm, dst_hbm, out_hbm,
             src_v, dst_v, rv, acc, sem):
        c = lax.axis_index("c")
        s = lax.axis_index("s")
        w = s * NC + c

        pltpu.sync_copy(src_hbm.at[w], src_v)
        pltpu.sync_copy(dst_hbm.at[w], dst_v)

        @pl.loop(0, K)
        def _(i):
            @pl.loop(0, D, step=L)
            def _(jj):
                rv[i, pl.ds(jj, L)] = jnp.zeros((L,), jnp.float32)

        @pl.loop(0, rows_per_tile, step=K)
        def _(r):
            pltpu.sync_copy(rv, acc.at[pl.ds(s * rows_per_tile + r, K)])

        plsc.subcore_barrier()

        @pl.loop(0, nchunks)
        def _(j):
            pltpu.async_copy(table_hbm.at[src_v.at[j]], rv, sem).wait()
            pltpu.sync_copy(rv, acc.at[dst_v.at[j]], add=True)

        plsc.subcore_barrier()
        pltpu.sync_copy(
            acc.at[pl.ds(s * rows_per_tile, rows_per_tile)],
            out_hbm.at[c, pl.ds(s * rows_per_tile, rows_per_tile)],
        )

    return pl.kernel(
        body,
        out_type=jax.ShapeDtypeStruct((NC, npad, D), jnp.float32),
        mesh=_MESH,
        scratch_types=[
            pltpu.VMEM((nchunks, K), jnp.int32),
            pltpu.VMEM((nchunks, K), jnp.int32),
            pltpu.VMEM((K, D), jnp.float32),
            pltpu.VMEM_SHARED((npad, D), jnp.float32),
            pltpu.SemaphoreType.DMA,
        ],
        compiler_params=_compiler_params(),
    )


def _tc_linear_relu(p, w, b, npad):
    br = 1024

    def body(p_ref, w_ref, b_ref, o_ref):
        x = p_ref[0] + p_ref[1]
        y = lax.dot_general(
            x, w_ref[...], (((1,), (1,)), ((), ())),
            preferred_element_type=jnp.float32,
        )
        o_ref[...] = jnp.maximum(y + b_ref[...], 0.0)

    return pl.pallas_call(
        body,
        grid=(npad // br,),
        in_specs=[
            pl.BlockSpec((NC, br, D), lambda i: (0, i, 0)),
            pl.BlockSpec((D, D), lambda i: (0, 0)),
            pl.BlockSpec((1, D), lambda i: (0, 0)),
        ],
        out_specs=pl.BlockSpec((br, D), lambda i: (i, 0)),
        out_shape=jax.ShapeDtypeStruct((npad, D), jnp.float32),
    )(p, w, b)


def kernel(cncpt_ids, edge_index, emb_table, W1, b1, W2, b2):
    n = cncpt_ids.shape[0]
    e = edge_index.shape[1]
    vocab = emb_table.shape[0]

    npad = -(-(n + 1) // (NS * K)) * (NS * K)
    nchunks = -(-e // (NW * K))
    epad = NW * nchunks * K

    src = edge_index[0].astype(jnp.int32)
    dst = edge_index[1].astype(jnp.int32)
    srcp = jnp.concatenate([src, jnp.zeros((epad - e,), jnp.int32)])
    dstp = jnp.concatenate([dst, jnp.full((epad - e,), n, jnp.int32)])
    srcp = srcp.reshape(NW, nchunks, K)
    dstp = dstp.reshape(NW, nchunks, K)

    fchunks = -(-n // (NW * K))
    frows = NW * fchunks * K
    cids = jnp.concatenate(
        [cncpt_ids.astype(jnp.int32), jnp.zeros((frows - n,), jnp.int32)])
    cids = cids.reshape(NW, fchunks, K)
    feats = _make_feats(vocab, fchunks)(emb_table, cids)

    p1 = _make_agg(nchunks, npad)(feats, srcp, dstp)
    h1 = _tc_linear_relu(p1, W1, b1.reshape(1, D), npad)

    p2 = _make_agg(nchunks, npad)(h1, srcp, dstp)
    h2 = _tc_linear_relu(p2, W2, b2.reshape(1, D), npad)
    return h2[:n]
